# baseline (speedup 1.0000x reference)
import functools

import jax
import jax.numpy as jnp
from jax.experimental import pallas as pl

PAD = 3000
H1 = 256
H2 = 128
BLK = 600


def _fused(n0, x_ref, fc0w_ref, fc0b_ref, wi_ref, bi_ref, wc_ref, bc_ref,
           wo_ref, bo_ref, fcw_ref, fcb_ref, hn_ref, y_ref):
    i = pl.program_id(0)
    row = i * BLK + jax.lax.broadcasted_iota(jnp.int32, (BLK, 1), 0)
    xb = jnp.where(row < n0, x_ref[:], 0.0)
    X = jax.nn.relu(
        jnp.dot(xb, fc0w_ref[:], preferred_element_type=jnp.float32)
        + fc0b_ref[:])
    I = jax.nn.sigmoid(
        jnp.dot(X, wi_ref[:], preferred_element_type=jnp.float32) + bi_ref[:])
    T = jnp.tanh(
        jnp.dot(X, wc_ref[:], preferred_element_type=jnp.float32) + bc_ref[:])
    O = jax.nn.sigmoid(
        jnp.dot(X, wo_ref[:], preferred_element_type=jnp.float32) + bo_ref[:])
    Hn = O * jnp.tanh(I * T)
    hn_ref[:] = Hn
    y_ref[:] = (jnp.sum(jax.nn.relu(Hn) * fcw_ref[:], axis=1, keepdims=True)
                + fcb_ref[:])


def kernel(x, edge_index, edge_weight, fc0_w, fc0_b,
           W_i, b_i, ci_t0, ci_t1, ci_b,
           W_f, b_f, cf_t0, cf_t1, cf_b,
           W_c, b_c, cc_t0, cc_t1, cc_b,
           W_o, b_o, co_t0, co_t1, co_b,
           fc_w, fc_b):
    n0 = x.shape[0]
    f_in = x.shape[1]
    fc0b = fc0_b.reshape(1, H1)
    bi = (b_i + ci_b).reshape(1, H2)
    bc = (b_c + cc_b).reshape(1, H2)
    bo = (b_o + co_b).reshape(1, H2)
    fcw = fc_w.reshape(1, H2)
    fcb = fc_b.reshape(1, 1)

    n_blk = PAD // BLK
    row_spec = pl.BlockSpec((BLK, f_in), lambda i: (i, 0))
    full = lambda a: pl.BlockSpec(a.shape, lambda i: (0, 0))

    hn, y = pl.pallas_call(
        functools.partial(_fused, n0),
        grid=(n_blk,),
        in_specs=[row_spec, full(fc0_w), full(fc0b), full(W_i), full(bi),
                  full(W_c), full(bc), full(W_o), full(bo), full(fcw),
                  full(fcb)],
        out_specs=[pl.BlockSpec((BLK, H2), lambda i: (i, 0)),
                   pl.BlockSpec((BLK, 1), lambda i: (i, 0))],
        out_shape=[
            jax.ShapeDtypeStruct((PAD, H2), jnp.float32),
            jax.ShapeDtypeStruct((PAD, 1), jnp.float32),
        ],
    )(x, fc0_w, fc0b, W_i, bi, W_c, bc, W_o, bo, fcw, fcb)

    return (y[:n0, 0], hn)


# speedup vs baseline: 1.0857x; 1.0857x over previous
import functools

import jax
import jax.numpy as jnp
from jax.experimental import pallas as pl

PAD = 3000
H1 = 256
H2 = 128
BLK = 1000


def _fused(n0, x_ref, fc0w_ref, fc0b_ref, wi_ref, bi_ref, wc_ref, bc_ref,
           wo_ref, bo_ref, fcw_ref, fcb_ref, hn_ref, y_ref):
    i = pl.program_id(0)
    row = i * BLK + jax.lax.broadcasted_iota(jnp.int32, (BLK, 1), 0)
    xb = jnp.where(row < n0, x_ref[:], 0.0)
    X = jax.nn.relu(
        jnp.dot(xb, fc0w_ref[:], preferred_element_type=jnp.float32)
        + fc0b_ref[:])
    I = jax.nn.sigmoid(
        jnp.dot(X, wi_ref[:], preferred_element_type=jnp.float32) + bi_ref[:])
    T = jnp.tanh(
        jnp.dot(X, wc_ref[:], preferred_element_type=jnp.float32) + bc_ref[:])
    O = jax.nn.sigmoid(
        jnp.dot(X, wo_ref[:], preferred_element_type=jnp.float32) + bo_ref[:])
    Hn = O * jnp.tanh(I * T)
    hn_ref[:] = Hn
    y_ref[:] = (jnp.sum(jax.nn.relu(Hn) * fcw_ref[:], axis=1, keepdims=True)
                + fcb_ref[:])


def kernel(x, edge_index, edge_weight, fc0_w, fc0_b,
           W_i, b_i, ci_t0, ci_t1, ci_b,
           W_f, b_f, cf_t0, cf_t1, cf_b,
           W_c, b_c, cc_t0, cc_t1, cc_b,
           W_o, b_o, co_t0, co_t1, co_b,
           fc_w, fc_b):
    n0 = x.shape[0]
    f_in = x.shape[1]
    fc0b = fc0_b.reshape(1, H1)
    bi = (b_i + ci_b).reshape(1, H2)
    bc = (b_c + cc_b).reshape(1, H2)
    bo = (b_o + co_b).reshape(1, H2)
    fcw = fc_w.reshape(1, H2)
    fcb = fc_b.reshape(1, 1)

    n_blk = PAD // BLK
    row_spec = pl.BlockSpec((BLK, f_in), lambda i: (i, 0))
    full = lambda a: pl.BlockSpec(a.shape, lambda i: (0, 0))

    hn, y = pl.pallas_call(
        functools.partial(_fused, n0),
        grid=(n_blk,),
        in_specs=[row_spec, full(fc0_w), full(fc0b), full(W_i), full(bi),
                  full(W_c), full(bc), full(W_o), full(bo), full(fcw),
                  full(fcb)],
        out_specs=[pl.BlockSpec((BLK, H2), lambda i: (i, 0)),
                   pl.BlockSpec((BLK, 1), lambda i: (i, 0))],
        out_shape=[
            jax.ShapeDtypeStruct((PAD, H2), jnp.float32),
            jax.ShapeDtypeStruct((PAD, 1), jnp.float32),
        ],
    )(x, fc0_w, fc0b, W_i, bi, W_c, bc, W_o, bo, fcw, fcb)

    return (y[:n0, 0], hn)
